# baseline (speedup 1.0000x reference)
import jax
import jax.numpy as jnp
from jax.experimental import pallas as pl
from jax.experimental.pallas import tpu as pltpu

NC = 21
C = 25
R = 6400
BETA = 0.5
NEG_POS_RATIO = 3.0


def _stage_a(yp_ref, yb_ref, anc_ref, alpha_ref, neg_ref, stats_ref, acc_ref):
    i = pl.program_id(0)
    nsteps = pl.num_programs(0)

    @pl.when(i == 0)
    def _init():
        acc_ref[0] = 0.0
        acc_ref[1] = 0.0
        acc_ref[2] = 0.0
        acc_ref[3] = 0.0

    xt = jnp.transpose(yp_ref[0])
    yt = jnp.transpose(yb_ref[0])
    anc = jnp.transpose(anc_ref[0])

    cp = xt[:NC]
    ch = yt[:NC]

    m = jnp.max(cp, axis=0, keepdims=True)
    se = jnp.sum(jnp.exp(cp - m), axis=0, keepdims=True)
    tl = jnp.sum(ch * cp, axis=0, keepdims=True)
    at = jnp.sum(ch * alpha_ref[:NC], axis=0, keepdims=True)
    logpt = tl - m - jnp.log(se)
    pt = jnp.exp(logpt)
    omp = 1.0 - pt
    loss = -(omp * omp) * (logpt * at)

    negm = ch[0:1] > 0.5
    posm = jnp.logical_not(negm)

    neg_ref[0] = jnp.where(negm, loss + 0.0, -1.0)

    a_xy = anc[0:2]
    wh_a = anc[2:4] - a_xy
    c_a = a_xy + 0.5 * wh_a
    dxy_p = c_a + xt[NC:NC + 2] * wh_a
    dwh_p = wh_a * jnp.exp(xt[NC + 2:NC + 4])
    lt_p = dxy_p - 0.5 * dwh_p
    rb_p = dxy_p + 0.5 * dwh_p
    dxy_t = c_a + yt[NC:NC + 2] * wh_a
    dwh_t = wh_a * jnp.exp(yt[NC + 2:NC + 4])
    lt_t = dxy_t - 0.5 * dwh_t
    rb_t = dxy_t + 0.5 * dwh_t
    whc = jnp.maximum(jnp.minimum(rb_p, rb_t) - jnp.maximum(lt_p, lt_t), 0.0)
    inter = whc[0:1] * whc[1:2]
    wh1 = jnp.maximum(rb_p - lt_p, 0.0)
    area1 = wh1[0:1] * wh1[1:2]
    wh2 = jnp.maximum(rb_t - lt_t, 0.0)
    area2 = wh2[0:1] * wh2[1:2]
    union = area1 + area2 - inter
    iou = inter / (union + 1e-8)

    zero = jnp.zeros_like(loss)
    acc_ref[0] = acc_ref[0] + jnp.sum(jnp.where(posm, loss, zero))
    acc_ref[1] = acc_ref[1] + jnp.sum(jnp.where(posm, 1.0 - iou, zero))
    acc_ref[2] = acc_ref[2] + jnp.sum(jnp.where(posm, 1.0, 0.0))
    acc_ref[3] = acc_ref[3] + jnp.sum(jnp.where(negm, 1.0, 0.0))

    @pl.when(i == nsteps - 1)
    def _fin():
        lane = jax.lax.broadcasted_iota(jnp.int32, (8, 128), 1)
        v = jnp.where(lane == 0, acc_ref[0],
                      jnp.where(lane == 1, acc_ref[1],
                                jnp.where(lane == 2, acc_ref[2], acc_ref[3])))
        stats_ref[...] = v


def _stage_b(neg_ref, stats_ref, out_ref):
    lane = jax.lax.broadcasted_iota(jnp.int32, (8, 128), 1)
    row = jax.lax.broadcasted_iota(jnp.int32, (8, 128), 0)
    stats = stats_ref[...]
    sel = (row == 0)

    def pick(j):
        return jnp.sum(jnp.where(sel & (lane == j), stats, 0.0))

    pos_sum = pick(0)
    reg_sum = pick(1)
    npos = pick(2)
    nneg = pick(3)
    k = jnp.minimum(nneg, NEG_POS_RATIO * npos)

    vals = neg_ref[...]
    bits = jax.lax.bitcast_convert_type(vals, jnp.int32)

    def body(_, carry):
        lo, hi = carry
        mid = lo + (hi - lo) // 2
        cnt = jnp.sum(jnp.where(bits >= mid, 1.0, 0.0))
        ok = cnt >= k
        return (jnp.where(ok, mid, lo), jnp.where(ok, hi, mid))

    lo, _ = jax.lax.fori_loop(
        0, 31, body, (jnp.int32(0), jnp.int32(0x7F800001)))
    gt = bits > lo
    cnt_gt = jnp.sum(jnp.where(gt, 1.0, 0.0))
    sum_gt = jnp.sum(jnp.where(gt, vals, 0.0))
    tval = jax.lax.bitcast_convert_type(lo, jnp.float32)
    hard_sum = sum_gt + (k - cnt_gt) * tval

    cls_neg = jnp.where(k > 0, hard_sum / jnp.maximum(k, 1.0), 0.0)
    cls_pos = jnp.where(npos > 0, pos_sum / jnp.maximum(npos, 1.0), 0.0)
    cls = jnp.where((nneg > 0) & (npos > 0), cls_pos + cls_neg, 0.0)
    reg = jnp.where(npos > 0, reg_sum / jnp.maximum(npos, 1.0), 0.0)
    total = cls + BETA * reg
    out_ref[...] = jnp.where(lane == 0, total,
                             jnp.where(lane == 1, cls, reg))


def kernel(y_pre, y_batch, anchor_boxes_xyxy, alpha):
    B, NA, _ = y_pre.shape
    N = B * NA
    NL = N // 128
    grid = N // R

    yp = y_pre.reshape(grid, R, C)
    yb = y_batch.reshape(grid, R, C)
    anct = jnp.tile(anchor_boxes_xyxy, (B, 1)).reshape(grid, R, 4)
    alpha_col = jnp.zeros((32, 1), jnp.float32).at[:NC, 0].set(alpha)

    neg, stats = pl.pallas_call(
        _stage_a,
        grid=(grid,),
        in_specs=[
            pl.BlockSpec((1, R, C), lambda i: (i, 0, 0)),
            pl.BlockSpec((1, R, C), lambda i: (i, 0, 0)),
            pl.BlockSpec((1, R, 4), lambda i: (i, 0, 0)),
            pl.BlockSpec((32, 1), lambda i: (0, 0)),
        ],
        out_specs=[
            pl.BlockSpec((1, 1, R), lambda i: (i, 0, 0)),
            pl.BlockSpec((8, 128), lambda i: (0, 0)),
        ],
        out_shape=[
            jax.ShapeDtypeStruct((grid, 1, R), jnp.float32),
            jax.ShapeDtypeStruct((8, 128), jnp.float32),
        ],
        scratch_shapes=[pltpu.SMEM((8,), jnp.float32)],
    )(yp, yb, anct, alpha_col)

    out = pl.pallas_call(
        _stage_b,
        in_specs=[
            pl.BlockSpec((NL, 128), lambda: (0, 0)),
            pl.BlockSpec((8, 128), lambda: (0, 0)),
        ],
        out_specs=pl.BlockSpec((8, 128), lambda: (0, 0)),
        out_shape=jax.ShapeDtypeStruct((8, 128), jnp.float32),
    )(neg.reshape(NL, 128), stats)

    return out[0, 0], out[0, 1], out[0, 2]


# speedup vs baseline: 1.3893x; 1.3893x over previous
import jax
import jax.numpy as jnp
from jax.experimental import pallas as pl
from jax.experimental.pallas import tpu as pltpu

NC = 21
C = 25
R = 4000
BETA = 0.5
NEG_POS_RATIO = 3.0


def _stage_a(yp_ref, yb_ref, anc_ref, alpha_ref, neg_ref, stats_ref, acc_ref):
    b = pl.program_id(0)
    j = pl.program_id(1)
    nb = pl.num_programs(0)
    nj = pl.num_programs(1)
    i = b * nj + j
    nsteps = nb * nj

    @pl.when(i == 0)
    def _init():
        acc_ref[0] = 0.0
        acc_ref[1] = 0.0
        acc_ref[2] = 0.0
        acc_ref[3] = 0.0

    xt = jnp.transpose(yp_ref[0])
    yt = jnp.transpose(yb_ref[0])
    anc = jnp.transpose(anc_ref[...])

    cp = xt[:NC]
    ch = yt[:NC]

    m = jnp.max(cp, axis=0, keepdims=True)
    se = jnp.sum(jnp.exp(cp - m), axis=0, keepdims=True)
    tl = jnp.sum(ch * cp, axis=0, keepdims=True)
    at = jnp.sum(ch * alpha_ref[:NC], axis=0, keepdims=True)
    logpt = tl - m - jnp.log(se)
    pt = jnp.exp(logpt)
    omp = 1.0 - pt
    loss = -(omp * omp) * (logpt * at)

    negm = ch[0:1] > 0.5
    posm = jnp.logical_not(negm)

    neg_ref[0, pl.ds(i % 8, 1)] = jnp.where(negm, loss + 0.0, -1.0)

    a_xy = anc[0:2]
    wh_a = anc[2:4] - a_xy
    c_a = a_xy + 0.5 * wh_a
    dxy_p = c_a + xt[NC:NC + 2] * wh_a
    dwh_p = wh_a * jnp.exp(xt[NC + 2:NC + 4])
    lt_p = dxy_p - 0.5 * dwh_p
    rb_p = dxy_p + 0.5 * dwh_p
    dxy_t = c_a + yt[NC:NC + 2] * wh_a
    dwh_t = wh_a * jnp.exp(yt[NC + 2:NC + 4])
    lt_t = dxy_t - 0.5 * dwh_t
    rb_t = dxy_t + 0.5 * dwh_t
    whc = jnp.maximum(jnp.minimum(rb_p, rb_t) - jnp.maximum(lt_p, lt_t), 0.0)
    inter = whc[0:1] * whc[1:2]
    wh1 = jnp.maximum(rb_p - lt_p, 0.0)
    area1 = wh1[0:1] * wh1[1:2]
    wh2 = jnp.maximum(rb_t - lt_t, 0.0)
    area2 = wh2[0:1] * wh2[1:2]
    union = area1 + area2 - inter
    iou = inter / (union + 1e-8)

    zero = jnp.zeros_like(loss)
    acc_ref[0] = acc_ref[0] + jnp.sum(jnp.where(posm, loss, zero))
    acc_ref[1] = acc_ref[1] + jnp.sum(jnp.where(posm, 1.0 - iou, zero))
    acc_ref[2] = acc_ref[2] + jnp.sum(jnp.where(posm, 1.0, 0.0))
    acc_ref[3] = acc_ref[3] + jnp.sum(jnp.where(negm, 1.0, 0.0))

    @pl.when(i == nsteps - 1)
    def _fin():
        lane = jax.lax.broadcasted_iota(jnp.int32, (8, 128), 1)
        v = jnp.where(lane == 0, acc_ref[0],
                      jnp.where(lane == 1, acc_ref[1],
                                jnp.where(lane == 2, acc_ref[2], acc_ref[3])))
        stats_ref[...] = v


def _stage_b(neg_ref, stats_ref, out_ref):
    lane = jax.lax.broadcasted_iota(jnp.int32, (8, 128), 1)
    row = jax.lax.broadcasted_iota(jnp.int32, (8, 128), 0)
    stats = stats_ref[...]
    sel = (row == 0)

    def pick(j):
        return jnp.sum(jnp.where(sel & (lane == j), stats, 0.0))

    pos_sum = pick(0)
    reg_sum = pick(1)
    npos = pick(2)
    nneg = pick(3)
    k = jnp.minimum(nneg, NEG_POS_RATIO * npos)

    vals = neg_ref[...]
    bits = jax.lax.bitcast_convert_type(vals, jnp.int32)

    def body(_, carry):
        lo, hi = carry
        mid = lo + (hi - lo) // 2
        cnt = jnp.sum(jnp.where(bits >= mid, 1.0, 0.0))
        ok = cnt >= k
        return (jnp.where(ok, mid, lo), jnp.where(ok, hi, mid))

    lo, _ = jax.lax.fori_loop(
        0, 31, body, (jnp.int32(0), jnp.int32(0x7F800001)))
    gt = bits > lo
    cnt_gt = jnp.sum(jnp.where(gt, 1.0, 0.0))
    sum_gt = jnp.sum(jnp.where(gt, vals, 0.0))
    tval = jax.lax.bitcast_convert_type(lo, jnp.float32)
    hard_sum = sum_gt + (k - cnt_gt) * tval

    cls_neg = jnp.where(k > 0, hard_sum / jnp.maximum(k, 1.0), 0.0)
    cls_pos = jnp.where(npos > 0, pos_sum / jnp.maximum(npos, 1.0), 0.0)
    cls = jnp.where((nneg > 0) & (npos > 0), cls_pos + cls_neg, 0.0)
    reg = jnp.where(npos > 0, reg_sum / jnp.maximum(npos, 1.0), 0.0)
    total = cls + BETA * reg
    out_ref[...] = jnp.where(lane == 0, total,
                             jnp.where(lane == 1, cls, reg))


def kernel(y_pre, y_batch, anchor_boxes_xyxy, alpha):
    B, NA, _ = y_pre.shape
    N = B * NA
    NL = N // 128
    nj = NA // R
    nsteps = B * nj

    alpha_col = jnp.zeros((32, 1), jnp.float32).at[:NC, 0].set(alpha)

    neg, stats = pl.pallas_call(
        _stage_a,
        grid=(B, nj),
        in_specs=[
            pl.BlockSpec((1, R, C), lambda b, j: (b, j, 0)),
            pl.BlockSpec((1, R, C), lambda b, j: (b, j, 0)),
            pl.BlockSpec((R, 4), lambda b, j: (j, 0)),
            pl.BlockSpec((32, 1), lambda b, j: (0, 0)),
        ],
        out_specs=[
            pl.BlockSpec((1, 8, R), lambda b, j: ((b * nj + j) // 8, 0, 0)),
            pl.BlockSpec((8, 128), lambda b, j: (0, 0)),
        ],
        out_shape=[
            jax.ShapeDtypeStruct((nsteps // 8, 8, R), jnp.float32),
            jax.ShapeDtypeStruct((8, 128), jnp.float32),
        ],
        scratch_shapes=[pltpu.SMEM((8,), jnp.float32)],
    )(y_pre, y_batch, anchor_boxes_xyxy, alpha_col)

    out = pl.pallas_call(
        _stage_b,
        in_specs=[
            pl.BlockSpec((NL, 128), lambda: (0, 0)),
            pl.BlockSpec((8, 128), lambda: (0, 0)),
        ],
        out_specs=pl.BlockSpec((8, 128), lambda: (0, 0)),
        out_shape=jax.ShapeDtypeStruct((8, 128), jnp.float32),
    )(neg.reshape(NL, 128), stats)

    return out[0, 0], out[0, 1], out[0, 2]


# speedup vs baseline: 1.7185x; 1.2370x over previous
import jax
import jax.numpy as jnp
from jax.experimental import pallas as pl
from jax.experimental.pallas import tpu as pltpu

NC = 21
C = 25
BB = 8
AB = 1000
R = BB * AB
BETA = 0.5
NEG_POS_RATIO = 3.0


def _stage_a(yp_ref, yb_ref, anc_ref, alpha_ref, neg_ref, stats_ref, acc_ref):
    tb = pl.program_id(0)
    j = pl.program_id(1)
    nj = pl.num_programs(1)
    i = tb * nj + j
    nsteps = pl.num_programs(0) * nj

    @pl.when(i == 0)
    def _init():
        acc_ref[0] = 0.0
        acc_ref[1] = 0.0
        acc_ref[2] = 0.0
        acc_ref[3] = 0.0

    xt = jnp.transpose(yp_ref[...].reshape(R, C))
    yt = jnp.transpose(yb_ref[...].reshape(R, C))
    a4 = anc_ref[:, 0, 0]
    anc = jnp.concatenate([a4] * BB, axis=1)

    cp = xt[:NC]
    ch = yt[:NC]

    m = jnp.max(cp, axis=0, keepdims=True)
    se = jnp.sum(jnp.exp(cp - m), axis=0, keepdims=True)
    tl = jnp.sum(ch * cp, axis=0, keepdims=True)
    at = jnp.sum(ch * alpha_ref[:NC], axis=0, keepdims=True)
    logpt = tl - m - jnp.log(se)
    pt = jnp.exp(logpt)
    omp = 1.0 - pt
    loss = -(omp * omp) * (logpt * at)

    negm = ch[0:1] > 0.5
    posm = jnp.logical_not(negm)

    neg_ref[0, pl.ds(i % 8, 1)] = jnp.where(negm, loss + 0.0, -1.0)

    a_xy = anc[0:2]
    wh_a = anc[2:4] - a_xy
    c_a = a_xy + 0.5 * wh_a
    dxy_p = c_a + xt[NC:NC + 2] * wh_a
    dwh_p = wh_a * jnp.exp(xt[NC + 2:NC + 4])
    lt_p = dxy_p - 0.5 * dwh_p
    rb_p = dxy_p + 0.5 * dwh_p
    dxy_t = c_a + yt[NC:NC + 2] * wh_a
    dwh_t = wh_a * jnp.exp(yt[NC + 2:NC + 4])
    lt_t = dxy_t - 0.5 * dwh_t
    rb_t = dxy_t + 0.5 * dwh_t
    whc = jnp.maximum(jnp.minimum(rb_p, rb_t) - jnp.maximum(lt_p, lt_t), 0.0)
    inter = whc[0:1] * whc[1:2]
    wh1 = jnp.maximum(rb_p - lt_p, 0.0)
    area1 = wh1[0:1] * wh1[1:2]
    wh2 = jnp.maximum(rb_t - lt_t, 0.0)
    area2 = wh2[0:1] * wh2[1:2]
    union = area1 + area2 - inter
    iou = inter / (union + 1e-8)

    zero = jnp.zeros_like(loss)
    acc_ref[0] = acc_ref[0] + jnp.sum(jnp.where(posm, loss, zero))
    acc_ref[1] = acc_ref[1] + jnp.sum(jnp.where(posm, 1.0 - iou, zero))
    acc_ref[2] = acc_ref[2] + jnp.sum(jnp.where(posm, 1.0, 0.0))
    acc_ref[3] = acc_ref[3] + jnp.sum(jnp.where(negm, 1.0, 0.0))

    @pl.when(i == nsteps - 1)
    def _fin():
        lane = jax.lax.broadcasted_iota(jnp.int32, (8, 128), 1)
        v = jnp.where(lane == 0, acc_ref[0],
                      jnp.where(lane == 1, acc_ref[1],
                                jnp.where(lane == 2, acc_ref[2], acc_ref[3])))
        stats_ref[...] = v


def _stage_b(neg_ref, stats_ref, out_ref):
    lane = jax.lax.broadcasted_iota(jnp.int32, (8, 128), 1)
    row = jax.lax.broadcasted_iota(jnp.int32, (8, 128), 0)
    stats = stats_ref[...]
    sel = (row == 0)

    def pick(j):
        return jnp.sum(jnp.where(sel & (lane == j), stats, 0.0))

    pos_sum = pick(0)
    reg_sum = pick(1)
    npos = pick(2)
    nneg = pick(3)
    k = jnp.minimum(nneg, NEG_POS_RATIO * npos)

    vals = neg_ref[...]
    bits = jax.lax.bitcast_convert_type(vals, jnp.int32)

    def body(_, carry):
        lo, hi = carry
        mid = lo + (hi - lo) // 2
        cnt = jnp.sum(jnp.where(bits >= mid, 1.0, 0.0))
        ok = cnt >= k
        return (jnp.where(ok, mid, lo), jnp.where(ok, hi, mid))

    lo, _ = jax.lax.fori_loop(
        0, 31, body, (jnp.int32(0), jnp.int32(0x7F800001)))
    gt = bits > lo
    cnt_gt = jnp.sum(jnp.where(gt, 1.0, 0.0))
    sum_gt = jnp.sum(jnp.where(gt, vals, 0.0))
    tval = jax.lax.bitcast_convert_type(lo, jnp.float32)
    hard_sum = sum_gt + (k - cnt_gt) * tval

    cls_neg = jnp.where(k > 0, hard_sum / jnp.maximum(k, 1.0), 0.0)
    cls_pos = jnp.where(npos > 0, pos_sum / jnp.maximum(npos, 1.0), 0.0)
    cls = jnp.where((nneg > 0) & (npos > 0), cls_pos + cls_neg, 0.0)
    reg = jnp.where(npos > 0, reg_sum / jnp.maximum(npos, 1.0), 0.0)
    total = cls + BETA * reg
    out_ref[...] = jnp.where(lane == 0, total,
                             jnp.where(lane == 1, cls, reg))


def kernel(y_pre, y_batch, anchor_boxes_xyxy, alpha):
    B, NA, _ = y_pre.shape
    N = B * NA
    ntb = B // BB
    nj = NA // AB
    nsteps = ntb * nj

    anct = jnp.transpose(anchor_boxes_xyxy).reshape(4, nj, 1, AB)
    alpha_col = jnp.zeros((32, 1), jnp.float32).at[:NC, 0].set(alpha)

    neg, stats = pl.pallas_call(
        _stage_a,
        grid=(ntb, nj),
        in_specs=[
            pl.BlockSpec((BB, AB, C), lambda tb, j: (tb, j, 0)),
            pl.BlockSpec((BB, AB, C), lambda tb, j: (tb, j, 0)),
            pl.BlockSpec((4, 1, 1, AB), lambda tb, j: (0, j, 0, 0)),
            pl.BlockSpec((32, 1), lambda tb, j: (0, 0)),
        ],
        out_specs=[
            pl.BlockSpec((1, 8, R),
                         lambda tb, j, _nj=nj: ((tb * _nj + j) // 8, 0, 0)),
            pl.BlockSpec((8, 128), lambda tb, j: (0, 0)),
        ],
        out_shape=[
            jax.ShapeDtypeStruct((nsteps // 8, 8, R), jnp.float32),
            jax.ShapeDtypeStruct((8, 128), jnp.float32),
        ],
        scratch_shapes=[pltpu.SMEM((8,), jnp.float32)],
    )(y_pre, y_batch, anct, alpha_col)

    out = pl.pallas_call(
        _stage_b,
        in_specs=[
            pl.BlockSpec((nsteps // 8, 8, R), lambda: (0, 0, 0)),
            pl.BlockSpec((8, 128), lambda: (0, 0)),
        ],
        out_specs=pl.BlockSpec((8, 128), lambda: (0, 0)),
        out_shape=jax.ShapeDtypeStruct((8, 128), jnp.float32),
    )(neg, stats)

    return out[0, 0], out[0, 1], out[0, 2]


# speedup vs baseline: 11.0325x; 6.4198x over previous
import jax
import jax.numpy as jnp
from jax.experimental import pallas as pl
from jax.experimental.pallas import tpu as pltpu

NC = 21
C = 25
BB = 8
CL = 2048
NA_TOTAL = 20000
BETA = 0.5
NEG_POS_RATIO = 3.0


def _stage_a(ypt_ref, ybt_ref, anc_ref, alpha_ref, neg_ref, stats_ref, acc_ref):
    tb = pl.program_id(0)
    j = pl.program_id(1)
    nj = pl.num_programs(1)
    i = tb * nj + j
    nsteps = pl.num_programs(0) * nj

    @pl.when(i == 0)
    def _init():
        acc_ref[...] = jnp.zeros_like(acc_ref)

    lanes = jax.lax.broadcasted_iota(jnp.int32, (BB, CL), 1)
    valid = (j * CL + lanes) < NA_TOTAL

    cp = [ypt_ref[c] for c in range(NC)]
    ch = [ybt_ref[c] for c in range(NC)]

    m = cp[0]
    for c in range(1, NC):
        m = jnp.maximum(m, cp[c])
    se = jnp.exp(cp[0] - m)
    tl = ch[0] * cp[0]
    at = ch[0] * alpha_ref[0]
    for c in range(1, NC):
        se = se + jnp.exp(cp[c] - m)
        tl = tl + ch[c] * cp[c]
        at = at + ch[c] * alpha_ref[c]
    logpt = tl - m - jnp.log(se)
    pt = jnp.exp(logpt)
    omp = 1.0 - pt
    loss = -(omp * omp) * (logpt * at)

    ch0 = ch[0]
    negm = (ch0 > 0.5) & valid
    posm = (ch0 < 0.5) & valid

    neg_ref[0, 0] = jnp.where(negm, loss + 0.0, -1.0)

    ax, ay, ax2, ay2 = (anc_ref[q:q + 1, :] for q in range(4))
    wa = ax2 - ax
    ha = ay2 - ay
    cx = ax + 0.5 * wa
    cy = ay + 0.5 * ha
    bpx, bpy, bpw, bph = (ypt_ref[NC + q] for q in range(4))
    bhx, bhy, bhw, bhh = (ybt_ref[NC + q] for q in range(4))
    dcxp = cx + bpx * wa
    dcyp = cy + bpy * ha
    dwp = wa * jnp.exp(bpw)
    dhp = ha * jnp.exp(bph)
    x1p = dcxp - 0.5 * dwp
    y1p = dcyp - 0.5 * dhp
    x2p = dcxp + 0.5 * dwp
    y2p = dcyp + 0.5 * dhp
    dcxt = cx + bhx * wa
    dcyt = cy + bhy * ha
    dwt = wa * jnp.exp(bhw)
    dht = ha * jnp.exp(bhh)
    x1t = dcxt - 0.5 * dwt
    y1t = dcyt - 0.5 * dht
    x2t = dcxt + 0.5 * dwt
    y2t = dcyt + 0.5 * dht
    iw = jnp.maximum(jnp.minimum(x2p, x2t) - jnp.maximum(x1p, x1t), 0.0)
    ih = jnp.maximum(jnp.minimum(y2p, y2t) - jnp.maximum(y1p, y1t), 0.0)
    inter = iw * ih
    a1 = jnp.maximum(x2p - x1p, 0.0) * jnp.maximum(y2p - y1p, 0.0)
    a2 = jnp.maximum(x2t - x1t, 0.0) * jnp.maximum(y2t - y1t, 0.0)
    union = a1 + a2 - inter
    iou = inter / (union + 1e-8)

    one = jnp.ones_like(loss)
    zero = jnp.zeros_like(loss)
    acc_ref[0] = acc_ref[0] + jnp.where(posm, loss, zero)
    acc_ref[1] = acc_ref[1] + jnp.where(posm, 1.0 - iou, zero)
    acc_ref[2] = acc_ref[2] + jnp.where(posm, one, zero)
    acc_ref[3] = acc_ref[3] + jnp.where(negm, one, zero)

    @pl.when(i == nsteps - 1)
    def _fin():
        lane = jax.lax.broadcasted_iota(jnp.int32, (8, 128), 1)
        v = jnp.where(lane == 0, jnp.sum(acc_ref[0]),
                      jnp.where(lane == 1, jnp.sum(acc_ref[1]),
                                jnp.where(lane == 2, jnp.sum(acc_ref[2]),
                                          jnp.sum(acc_ref[3]))))
        stats_ref[...] = v


def _stage_b(neg_ref, stats_ref, out_ref):
    lane = jax.lax.broadcasted_iota(jnp.int32, (8, 128), 1)
    row = jax.lax.broadcasted_iota(jnp.int32, (8, 128), 0)
    stats = stats_ref[...]
    sel = (row == 0)

    def pick(j):
        return jnp.sum(jnp.where(sel & (lane == j), stats, 0.0))

    pos_sum = pick(0)
    reg_sum = pick(1)
    npos = pick(2)
    nneg = pick(3)
    k = jnp.minimum(nneg, NEG_POS_RATIO * npos)

    vals = neg_ref[...]
    bits = jax.lax.bitcast_convert_type(vals, jnp.int32)

    def body(_, carry):
        lo, hi = carry
        mid = lo + (hi - lo) // 2
        cnt = jnp.sum(jnp.where(bits >= mid, 1.0, 0.0))
        ok = cnt >= k
        return (jnp.where(ok, mid, lo), jnp.where(ok, hi, mid))

    lo, _ = jax.lax.fori_loop(
        0, 31, body, (jnp.int32(0), jnp.int32(0x7F800001)))
    gt = bits > lo
    cnt_gt = jnp.sum(jnp.where(gt, 1.0, 0.0))
    sum_gt = jnp.sum(jnp.where(gt, vals, 0.0))
    tval = jax.lax.bitcast_convert_type(lo, jnp.float32)
    hard_sum = sum_gt + (k - cnt_gt) * tval

    cls_neg = jnp.where(k > 0, hard_sum / jnp.maximum(k, 1.0), 0.0)
    cls_pos = jnp.where(npos > 0, pos_sum / jnp.maximum(npos, 1.0), 0.0)
    cls = jnp.where((nneg > 0) & (npos > 0), cls_pos + cls_neg, 0.0)
    reg = jnp.where(npos > 0, reg_sum / jnp.maximum(npos, 1.0), 0.0)
    total = cls + BETA * reg
    out_ref[...] = jnp.where(lane == 0, total,
                             jnp.where(lane == 1, cls, reg))


def kernel(y_pre, y_batch, anchor_boxes_xyxy, alpha):
    B, NA, _ = y_pre.shape
    ntb = B // BB
    nj = (NA + CL - 1) // CL
    nsteps = ntb * nj

    ypt = jnp.transpose(y_pre, (2, 0, 1))
    ybt = jnp.transpose(y_batch, (2, 0, 1))
    anct = jnp.transpose(anchor_boxes_xyxy)

    neg, stats = pl.pallas_call(
        _stage_a,
        grid=(ntb, nj),
        in_specs=[
            pl.BlockSpec((C, BB, CL), lambda tb, j: (0, tb, j)),
            pl.BlockSpec((C, BB, CL), lambda tb, j: (0, tb, j)),
            pl.BlockSpec((4, CL), lambda tb, j: (0, j)),
            pl.BlockSpec(memory_space=pltpu.SMEM),
        ],
        out_specs=[
            pl.BlockSpec((1, 1, BB, CL), lambda tb, j: (tb, j, 0, 0)),
            pl.BlockSpec((8, 128), lambda tb, j: (0, 0)),
        ],
        out_shape=[
            jax.ShapeDtypeStruct((ntb, nj, BB, CL), jnp.float32),
            jax.ShapeDtypeStruct((8, 128), jnp.float32),
        ],
        scratch_shapes=[pltpu.VMEM((4, BB, CL), jnp.float32)],
    )(ypt, ybt, anct, alpha)

    nl = ntb * nj * BB * CL // 128
    out = pl.pallas_call(
        _stage_b,
        in_specs=[
            pl.BlockSpec((nl, 128), lambda: (0, 0)),
            pl.BlockSpec((8, 128), lambda: (0, 0)),
        ],
        out_specs=pl.BlockSpec((8, 128), lambda: (0, 0)),
        out_shape=jax.ShapeDtypeStruct((8, 128), jnp.float32),
    )(neg.reshape(nl, 128), stats)

    return out[0, 0], out[0, 1], out[0, 2]


# speedup vs baseline: 12.2676x; 1.1120x over previous
import jax
import jax.numpy as jnp
from jax.experimental import pallas as pl
from jax.experimental.pallas import tpu as pltpu

NC = 21
C = 25
BB = 8
CL = 2048
NA_TOTAL = 20000
BETA = 0.5
NEG_POS_RATIO = 3.0


def _stage_a(ypt_ref, ybt_ref, anc_ref, alpha_ref, out_ref, acc_ref, negbuf_ref):
    tb = pl.program_id(0)
    j = pl.program_id(1)
    nj = pl.num_programs(1)
    i = tb * nj + j
    nsteps = pl.num_programs(0) * nj

    @pl.when(i == 0)
    def _init():
        acc_ref[...] = jnp.zeros_like(acc_ref)

    lanes = jax.lax.broadcasted_iota(jnp.int32, (BB, CL), 1)
    valid = (j * CL + lanes) < NA_TOTAL

    cp = [ypt_ref[c] for c in range(NC)]
    ch = [ybt_ref[c] for c in range(NC)]

    m = cp[0]
    for c in range(1, NC):
        m = jnp.maximum(m, cp[c])
    se = jnp.exp(cp[0] - m)
    tl = ch[0] * cp[0]
    at = ch[0] * alpha_ref[0]
    for c in range(1, NC):
        se = se + jnp.exp(cp[c] - m)
        tl = tl + ch[c] * cp[c]
        at = at + ch[c] * alpha_ref[c]
    logpt = tl - m - jnp.log(se)
    pt = jnp.exp(logpt)
    omp = 1.0 - pt
    loss = -(omp * omp) * (logpt * at)

    ch0 = ch[0]
    negm = (ch0 > 0.5) & valid
    posm = (ch0 < 0.5) & valid

    negbuf_ref[i] = jnp.where(negm, loss + 0.0, -1.0)

    ax, ay, ax2, ay2 = (anc_ref[q:q + 1, :] for q in range(4))
    wa = ax2 - ax
    ha = ay2 - ay
    cx = ax + 0.5 * wa
    cy = ay + 0.5 * ha
    bpx, bpy, bpw, bph = (ypt_ref[NC + q] for q in range(4))
    bhx, bhy, bhw, bhh = (ybt_ref[NC + q] for q in range(4))
    dcxp = cx + bpx * wa
    dcyp = cy + bpy * ha
    dwp = wa * jnp.exp(bpw)
    dhp = ha * jnp.exp(bph)
    x1p = dcxp - 0.5 * dwp
    y1p = dcyp - 0.5 * dhp
    x2p = dcxp + 0.5 * dwp
    y2p = dcyp + 0.5 * dhp
    dcxt = cx + bhx * wa
    dcyt = cy + bhy * ha
    dwt = wa * jnp.exp(bhw)
    dht = ha * jnp.exp(bhh)
    x1t = dcxt - 0.5 * dwt
    y1t = dcyt - 0.5 * dht
    x2t = dcxt + 0.5 * dwt
    y2t = dcyt + 0.5 * dht
    iw = jnp.maximum(jnp.minimum(x2p, x2t) - jnp.maximum(x1p, x1t), 0.0)
    ih = jnp.maximum(jnp.minimum(y2p, y2t) - jnp.maximum(y1p, y1t), 0.0)
    inter = iw * ih
    a1 = jnp.maximum(x2p - x1p, 0.0) * jnp.maximum(y2p - y1p, 0.0)
    a2 = jnp.maximum(x2t - x1t, 0.0) * jnp.maximum(y2t - y1t, 0.0)
    union = a1 + a2 - inter
    iou = inter / (union + 1e-8)

    one = jnp.ones_like(loss)
    zero = jnp.zeros_like(loss)
    acc_ref[0] = acc_ref[0] + jnp.where(posm, loss, zero)
    acc_ref[1] = acc_ref[1] + jnp.where(posm, 1.0 - iou, zero)
    acc_ref[2] = acc_ref[2] + jnp.where(posm, one, zero)
    acc_ref[3] = acc_ref[3] + jnp.where(negm, one, zero)

    @pl.when(i == nsteps - 1)
    def _fin():
        pos_sum = jnp.sum(acc_ref[0])
        reg_sum = jnp.sum(acc_ref[1])
        npos = jnp.sum(acc_ref[2])
        nneg = jnp.sum(acc_ref[3])
        k = jnp.minimum(nneg, NEG_POS_RATIO * npos)

        vals = negbuf_ref[...]
        bits = jax.lax.bitcast_convert_type(vals, jnp.int32)

        def body(_, carry):
            lo, hi = carry
            mid = lo + (hi - lo) // 2
            cnt = jnp.sum(jnp.where(bits >= mid, 1.0, 0.0))
            ok = cnt >= k
            return (jnp.where(ok, mid, lo), jnp.where(ok, hi, mid))

        lo, _ = jax.lax.fori_loop(
            0, 31, body, (jnp.int32(0), jnp.int32(0x7F800001)))
        gt = bits > lo
        cnt_gt = jnp.sum(jnp.where(gt, 1.0, 0.0))
        sum_gt = jnp.sum(jnp.where(gt, vals, 0.0))
        tval = jax.lax.bitcast_convert_type(lo, jnp.float32)
        hard_sum = sum_gt + (k - cnt_gt) * tval

        cls_neg = jnp.where(k > 0, hard_sum / jnp.maximum(k, 1.0), 0.0)
        cls_pos = jnp.where(npos > 0, pos_sum / jnp.maximum(npos, 1.0), 0.0)
        cls = jnp.where((nneg > 0) & (npos > 0), cls_pos + cls_neg, 0.0)
        reg = jnp.where(npos > 0, reg_sum / jnp.maximum(npos, 1.0), 0.0)
        total = cls + BETA * reg
        lane = jax.lax.broadcasted_iota(jnp.int32, (8, 128), 1)
        out_ref[...] = jnp.where(lane == 0, total,
                                 jnp.where(lane == 1, cls, reg))


def kernel(y_pre, y_batch, anchor_boxes_xyxy, alpha):
    B, NA, _ = y_pre.shape
    ntb = B // BB
    nj = (NA + CL - 1) // CL
    nsteps = ntb * nj

    ypt = jnp.transpose(y_pre, (2, 0, 1))
    ybt = jnp.transpose(y_batch, (2, 0, 1))
    anct = jnp.transpose(anchor_boxes_xyxy)

    out = pl.pallas_call(
        _stage_a,
        grid=(ntb, nj),
        in_specs=[
            pl.BlockSpec((C, BB, CL), lambda tb, j: (0, tb, j)),
            pl.BlockSpec((C, BB, CL), lambda tb, j: (0, tb, j)),
            pl.BlockSpec((4, CL), lambda tb, j: (0, j)),
            pl.BlockSpec(memory_space=pltpu.SMEM),
        ],
        out_specs=pl.BlockSpec((8, 128), lambda tb, j: (0, 0)),
        out_shape=jax.ShapeDtypeStruct((8, 128), jnp.float32),
        scratch_shapes=[
            pltpu.VMEM((4, BB, CL), jnp.float32),
            pltpu.VMEM((ntb * nj, BB, CL), jnp.float32),
        ],
    )(ypt, ybt, anct, alpha)

    return out[0, 0], out[0, 1], out[0, 2]


# speedup vs baseline: 13.3592x; 1.0890x over previous
import jax
import jax.numpy as jnp
from jax.experimental import pallas as pl
from jax.experimental.pallas import tpu as pltpu

NC = 21
C = 25
BB = 8
CL = 4096
NA_TOTAL = 20000
BETA = 0.5
NEG_POS_RATIO = 3.0


def _stage_a(ypt_ref, ybt_ref, anc_ref, alpha_ref, out_ref, acc_ref, negbuf_ref):
    tb = pl.program_id(0)
    j = pl.program_id(1)
    nj = pl.num_programs(1)
    i = tb * nj + j
    nsteps = pl.num_programs(0) * nj

    @pl.when(i == 0)
    def _init():
        acc_ref[...] = jnp.zeros_like(acc_ref)

    lanes = jax.lax.broadcasted_iota(jnp.int32, (BB, CL), 1)
    valid = (j * CL + lanes) < NA_TOTAL

    cp = [ypt_ref[c] for c in range(NC)]
    ch = [ybt_ref[c] for c in range(NC)]

    def _tree(xs, op):
        while len(xs) > 1:
            tail = [xs[-1]] if len(xs) % 2 else []
            xs = [op(xs[p], xs[p + 1]) for p in range(0, len(xs) - 1, 2)] + tail
        return xs[0]

    add = lambda a, b: a + b
    m = _tree(list(cp), jnp.maximum)
    se = _tree([jnp.exp(cp[c] - m) for c in range(NC)], add)
    tl = _tree([ch[c] * cp[c] for c in range(NC)], add)
    at = _tree([ch[c] * alpha_ref[c] for c in range(NC)], add)
    logpt = tl - m - jnp.log(se)
    pt = jnp.exp(logpt)
    omp = 1.0 - pt
    loss = -(omp * omp) * (logpt * at)

    ch0 = ch[0]
    negm = (ch0 > 0.5) & valid
    posm = (ch0 < 0.5) & valid

    negbuf_ref[i] = jnp.where(negm, loss + 0.0, -1.0)

    ax, ay, ax2, ay2 = (anc_ref[q:q + 1, :] for q in range(4))
    wa = ax2 - ax
    ha = ay2 - ay
    cx = ax + 0.5 * wa
    cy = ay + 0.5 * ha
    bpx, bpy, bpw, bph = (ypt_ref[NC + q] for q in range(4))
    bhx, bhy, bhw, bhh = (ybt_ref[NC + q] for q in range(4))
    dcxp = cx + bpx * wa
    dcyp = cy + bpy * ha
    dwp = wa * jnp.exp(bpw)
    dhp = ha * jnp.exp(bph)
    x1p = dcxp - 0.5 * dwp
    y1p = dcyp - 0.5 * dhp
    x2p = dcxp + 0.5 * dwp
    y2p = dcyp + 0.5 * dhp
    dcxt = cx + bhx * wa
    dcyt = cy + bhy * ha
    dwt = wa * jnp.exp(bhw)
    dht = ha * jnp.exp(bhh)
    x1t = dcxt - 0.5 * dwt
    y1t = dcyt - 0.5 * dht
    x2t = dcxt + 0.5 * dwt
    y2t = dcyt + 0.5 * dht
    iw = jnp.maximum(jnp.minimum(x2p, x2t) - jnp.maximum(x1p, x1t), 0.0)
    ih = jnp.maximum(jnp.minimum(y2p, y2t) - jnp.maximum(y1p, y1t), 0.0)
    inter = iw * ih
    a1 = jnp.maximum(x2p - x1p, 0.0) * jnp.maximum(y2p - y1p, 0.0)
    a2 = jnp.maximum(x2t - x1t, 0.0) * jnp.maximum(y2t - y1t, 0.0)
    union = a1 + a2 - inter
    iou = inter / (union + 1e-8)

    one = jnp.ones_like(loss)
    zero = jnp.zeros_like(loss)
    acc_ref[0] = acc_ref[0] + jnp.where(posm, loss, zero)
    acc_ref[1] = acc_ref[1] + jnp.where(posm, 1.0 - iou, zero)
    acc_ref[2] = acc_ref[2] + jnp.where(posm, one, zero)
    acc_ref[3] = acc_ref[3] + jnp.where(negm, one, zero)

    @pl.when(i == nsteps - 1)
    def _fin():
        pos_sum = jnp.sum(acc_ref[0])
        reg_sum = jnp.sum(acc_ref[1])
        npos = jnp.sum(acc_ref[2])
        nneg = jnp.sum(acc_ref[3])
        k = jnp.minimum(nneg, NEG_POS_RATIO * npos)

        vals = negbuf_ref[...]
        bits = jax.lax.bitcast_convert_type(vals, jnp.int32)

        def body(_, carry):
            lo, hi = carry
            mid = lo + (hi - lo) // 2
            cnt = jnp.sum(jnp.where(bits >= mid, 1.0, 0.0))
            ok = cnt >= k
            return (jnp.where(ok, mid, lo), jnp.where(ok, hi, mid))

        lo, _ = jax.lax.fori_loop(
            0, 31, body, (jnp.int32(0), jnp.int32(0x7F800001)))
        gt = bits > lo
        cnt_gt = jnp.sum(jnp.where(gt, 1.0, 0.0))
        sum_gt = jnp.sum(jnp.where(gt, vals, 0.0))
        tval = jax.lax.bitcast_convert_type(lo, jnp.float32)
        hard_sum = sum_gt + (k - cnt_gt) * tval

        cls_neg = jnp.where(k > 0, hard_sum / jnp.maximum(k, 1.0), 0.0)
        cls_pos = jnp.where(npos > 0, pos_sum / jnp.maximum(npos, 1.0), 0.0)
        cls = jnp.where((nneg > 0) & (npos > 0), cls_pos + cls_neg, 0.0)
        reg = jnp.where(npos > 0, reg_sum / jnp.maximum(npos, 1.0), 0.0)
        total = cls + BETA * reg
        lane = jax.lax.broadcasted_iota(jnp.int32, (8, 128), 1)
        out_ref[...] = jnp.where(lane == 0, total,
                                 jnp.where(lane == 1, cls, reg))


def kernel(y_pre, y_batch, anchor_boxes_xyxy, alpha):
    B, NA, _ = y_pre.shape
    ntb = B // BB
    nj = (NA + CL - 1) // CL
    nsteps = ntb * nj

    ypt = jnp.transpose(y_pre, (2, 0, 1))
    ybt = jnp.transpose(y_batch, (2, 0, 1))
    anct = jnp.transpose(anchor_boxes_xyxy)

    out = pl.pallas_call(
        _stage_a,
        grid=(ntb, nj),
        in_specs=[
            pl.BlockSpec((C, BB, CL), lambda tb, j: (0, tb, j)),
            pl.BlockSpec((C, BB, CL), lambda tb, j: (0, tb, j)),
            pl.BlockSpec((4, CL), lambda tb, j: (0, j)),
            pl.BlockSpec(memory_space=pltpu.SMEM),
        ],
        out_specs=pl.BlockSpec((8, 128), lambda tb, j: (0, 0)),
        out_shape=jax.ShapeDtypeStruct((8, 128), jnp.float32),
        scratch_shapes=[
            pltpu.VMEM((4, BB, CL), jnp.float32),
            pltpu.VMEM((ntb * nj, BB, CL), jnp.float32),
        ],
    )(ypt, ybt, anct, alpha)

    return out[0, 0], out[0, 1], out[0, 2]
